# baseline (speedup 1.0000x reference)
import jax
import jax.numpy as jnp
from jax import lax
from jax.experimental import pallas as pl
from jax.experimental.pallas import tpu as pltpu
from jax.experimental.pallas import tpu_sc as plsc

N = 65536
R = 8192
B = 256
L = 16
NW = 32
ROWS_PER_W = N // NW
C = 64
NCHUNK = ROWS_PER_W // C

PRUNE_THRESHOLD = 0.05


def _sc_body(acts, src, dst, tro, ema, act, coef,
             o_tro, o_act, o_ema, o_gain,
             idx_s, idx_d,
             a0, b0, a1, b1, co_buf,
             tro_buf, ema_buf, act_buf,
             ot_buf, oa_buf, oe_buf, og_buf,
             coef_v, sem_a0, sem_b0, sem_a1, sem_b1, sem_in):
    wid = lax.axis_index("s") * 2 + lax.axis_index("c")
    base = wid * ROWS_PER_W

    ci = pltpu.async_copy(src.at[pl.ds(base, ROWS_PER_W)], idx_s, sem_in)
    cd = pltpu.async_copy(dst.at[pl.ds(base, ROWS_PER_W)], idx_d, sem_in)
    ci.wait()
    cd.wait()

    bufs = ((a0, b0, sem_a0, sem_b0), (a1, b1, sem_a1, sem_b1))

    def start(g, slot):
        a_b, b_b, s_a, s_b = bufs[slot]
        off = g * C
        pltpu.async_copy(acts.at[idx_s.at[pl.ds(off, C)]], a_b, s_a)
        pltpu.async_copy(acts.at[idx_d.at[pl.ds(off, C)]], b_b, s_b)

    def wait(g, slot):
        a_b, b_b, s_a, s_b = bufs[slot]
        off = g * C
        pltpu.make_async_copy(acts.at[idx_s.at[pl.ds(off, C)]], a_b,
                              s_a).wait()
        pltpu.make_async_copy(acts.at[idx_d.at[pl.ds(off, C)]], b_b,
                              s_b).wait()

    lane_ids = lax.iota(jnp.int32, L)
    lane0 = lane_ids == 0

    def compute(g, slot):
        a_b, b_b = bufs[slot][0], bufs[slot][1]
        off = g * C

        @plsc.parallel_loop(0, C, unroll=2)
        def row_body(r):
            acc = a_b[r, pl.ds(0, L)] * b_b[r, pl.ds(0, L)]
            for k in range(1, B // L):
                acc = acc + (a_b[r, pl.ds(k * L, L)]
                             * b_b[r, pl.ds(k * L, L)])
            s = jnp.broadcast_to(jnp.sum(acc), (L,))
            plsc.store_scatter(co_buf, [jnp.broadcast_to(r, (L,))], s,
                               mask=lane0)

        @plsc.parallel_loop(0, C // L)
        def group_body(t):
            sl = pl.ds(off + t * L, L)
            co = jnp.clip(co_buf[pl.ds(t * L, L)] * (1.0 / B), 0.0, 1.0)
            new_ema = 0.95 * ema_buf[sl] + 0.05 * co
            new_t = jnp.clip(tro_buf[sl] + (c0 + c1 * new_ema), 0.0, 1.0)
            ac = act_buf[sl]
            pruned = new_t < PRUNE_THRESHOLD
            resprout = (new_t > PRUNE_THRESHOLD * 2.0) & (ac == 0.0)
            new_a = jnp.where(pruned, 0.0, jnp.where(resprout, 1.0, ac))
            ot_buf[sl] = new_t
            oa_buf[sl] = new_a
            oe_buf[sl] = new_ema
            og_buf[sl] = new_a * (0.2 + 1.6 * new_t)

    start(0, 0)

    c1_ = pltpu.async_copy(tro.at[pl.ds(base, ROWS_PER_W)], tro_buf,
                           sem_in)
    c2_ = pltpu.async_copy(ema.at[pl.ds(base, ROWS_PER_W)], ema_buf,
                           sem_in)
    c3_ = pltpu.async_copy(act.at[pl.ds(base, ROWS_PER_W)], act_buf,
                           sem_in)
    c4_ = pltpu.async_copy(coef, coef_v, sem_in)
    c1_.wait()
    c2_.wait()
    c3_.wait()
    c4_.wait()
    c0 = coef_v[0, pl.ds(0, L)]
    c1 = coef_v[1, pl.ds(0, L)]

    def pair_body(i, _):
        g = i * 2
        for s in range(2):
            gg = g + s

            @pl.when(gg + 1 < NCHUNK)
            def _():
                start(gg + 1, (s + 1) % 2)

            wait(gg, s)
            compute(gg, s)
        return 0

    lax.fori_loop(0, NCHUNK // 2, pair_body, 0)

    pltpu.sync_copy(ot_buf, o_tro.at[pl.ds(base, ROWS_PER_W)])
    pltpu.sync_copy(oa_buf, o_act.at[pl.ds(base, ROWS_PER_W)])
    pltpu.sync_copy(oe_buf, o_ema.at[pl.ds(base, ROWS_PER_W)])
    pltpu.sync_copy(og_buf, o_gain.at[pl.ds(base, ROWS_PER_W)])


@jax.jit
def _run(activities, src_idx, dst_idx, trophic, ema_coact, active, coef):
    f32 = jnp.float32
    out = jax.ShapeDtypeStruct((N,), f32)
    k = pl.kernel(
        _sc_body,
        out_type=(out, out, out, out),
        mesh=plsc.VectorSubcoreMesh(core_axis_name="c", subcore_axis_name="s"),
        compiler_params=pltpu.CompilerParams(needs_layout_passes=False),
        scratch_types=(
            pltpu.VMEM((ROWS_PER_W,), jnp.int32),
            pltpu.VMEM((ROWS_PER_W,), jnp.int32),
            pltpu.VMEM((C, B), f32),
            pltpu.VMEM((C, B), f32),
            pltpu.VMEM((C, B), f32),
            pltpu.VMEM((C, B), f32),
            pltpu.VMEM((C,), f32),
            pltpu.VMEM((ROWS_PER_W,), f32),
            pltpu.VMEM((ROWS_PER_W,), f32),
            pltpu.VMEM((ROWS_PER_W,), f32),
            pltpu.VMEM((ROWS_PER_W,), f32),
            pltpu.VMEM((ROWS_PER_W,), f32),
            pltpu.VMEM((ROWS_PER_W,), f32),
            pltpu.VMEM((ROWS_PER_W,), f32),
            pltpu.VMEM((2, L), f32),
            pltpu.SemaphoreType.DMA,
            pltpu.SemaphoreType.DMA,
            pltpu.SemaphoreType.DMA,
            pltpu.SemaphoreType.DMA,
            pltpu.SemaphoreType.DMA,
        ),
    )
    return k(activities, src_idx, dst_idx, trophic, ema_coact, active, coef)


def kernel(activities, src_idx, dst_idx, trophic, ema_coact, active,
           bdnf, ngf):
    bdnf_eff = jnp.clip(bdnf[0] * 0.05, 0.0, 0.05)
    ngf_eff = jnp.clip(ngf[0] * 0.01, 0.0, 0.01)
    g = bdnf_eff + 0.005
    c0 = 0.1 * g - (ngf_eff + 0.003)
    c1 = g + 0.001
    coef = jnp.stack([jnp.broadcast_to(c0, (L,)),
                      jnp.broadcast_to(c1, (L,))]).astype(jnp.float32)
    return _run(activities, src_idx, dst_idx, trophic, ema_coact, active,
                coef)


# speedup vs baseline: 1.0830x; 1.0830x over previous
import jax
import jax.numpy as jnp
from jax import lax
from jax.experimental import pallas as pl
from jax.experimental.pallas import tpu as pltpu
from jax.experimental.pallas import tpu_sc as plsc

N = 65536
R = 8192
B = 256
L = 16
NW = 32
ROWS_PER_W = N // NW
C = 32
NCHUNK = ROWS_PER_W // C

PRUNE_THRESHOLD = 0.05


def _sc_body(acts, src, dst, tro, ema, act, coef,
             o_tro, o_act, o_ema, o_gain,
             idx_s, idx_d,
             a0, b0, a1, b1, a2, b2, a3, b3, co_buf,
             tro_buf, ema_buf, act_buf,
             ot_buf, oa_buf, oe_buf, og_buf,
             coef_v, sem_a0, sem_b0, sem_a1, sem_b1,
             sem_a2, sem_b2, sem_a3, sem_b3, sem_in):
    wid = lax.axis_index("s") * 2 + lax.axis_index("c")
    base = wid * ROWS_PER_W

    ci = pltpu.async_copy(src.at[pl.ds(base, ROWS_PER_W)], idx_s, sem_in)
    cd = pltpu.async_copy(dst.at[pl.ds(base, ROWS_PER_W)], idx_d, sem_in)
    ci.wait()
    cd.wait()

    bufs = ((a0, b0, sem_a0, sem_b0), (a1, b1, sem_a1, sem_b1),
            (a2, b2, sem_a2, sem_b2), (a3, b3, sem_a3, sem_b3))
    NSLOT = len(bufs)

    def start(g, slot):
        a_b, b_b, s_a, s_b = bufs[slot]
        off = g * C
        pltpu.async_copy(acts.at[idx_s.at[pl.ds(off, C)]], a_b, s_a)
        pltpu.async_copy(acts.at[idx_d.at[pl.ds(off, C)]], b_b, s_b)

    def wait(g, slot):
        a_b, b_b, s_a, s_b = bufs[slot]
        off = g * C
        pltpu.make_async_copy(acts.at[idx_s.at[pl.ds(off, C)]], a_b,
                              s_a).wait()
        pltpu.make_async_copy(acts.at[idx_d.at[pl.ds(off, C)]], b_b,
                              s_b).wait()

    lane_ids = lax.iota(jnp.int32, L)
    lane0 = lane_ids == 0

    def compute(g, slot):
        a_b, b_b = bufs[slot][0], bufs[slot][1]
        off = g * C

        @plsc.parallel_loop(0, C)
        def row_body(r):
            acc = a_b[r, pl.ds(0, L)] * b_b[r, pl.ds(0, L)]
            for k in range(1, B // L):
                acc = acc + (a_b[r, pl.ds(k * L, L)]
                             * b_b[r, pl.ds(k * L, L)])
            s = jnp.broadcast_to(jnp.sum(acc), (L,))
            plsc.store_scatter(co_buf, [jnp.broadcast_to(r, (L,))], s,
                               mask=lane0)

        @plsc.parallel_loop(0, C // L)
        def group_body(t):
            sl = pl.ds(off + t * L, L)
            co = jnp.clip(co_buf[pl.ds(t * L, L)] * (1.0 / B), 0.0, 1.0)
            new_ema = 0.95 * ema_buf[sl] + 0.05 * co
            new_t = jnp.clip(tro_buf[sl] + (c0 + c1 * new_ema), 0.0, 1.0)
            ac = act_buf[sl]
            pruned = new_t < PRUNE_THRESHOLD
            resprout = (new_t > PRUNE_THRESHOLD * 2.0) & (ac == 0.0)
            new_a = jnp.where(pruned, 0.0, jnp.where(resprout, 1.0, ac))
            ot_buf[sl] = new_t
            oa_buf[sl] = new_a
            oe_buf[sl] = new_ema
            og_buf[sl] = new_a * (0.2 + 1.6 * new_t)

    start(0, 0)
    start(1, 1)
    start(2, 2)

    c1_ = pltpu.async_copy(tro.at[pl.ds(base, ROWS_PER_W)], tro_buf,
                           sem_in)
    c2_ = pltpu.async_copy(ema.at[pl.ds(base, ROWS_PER_W)], ema_buf,
                           sem_in)
    c3_ = pltpu.async_copy(act.at[pl.ds(base, ROWS_PER_W)], act_buf,
                           sem_in)
    c4_ = pltpu.async_copy(coef, coef_v, sem_in)
    c1_.wait()
    c2_.wait()
    c3_.wait()
    c4_.wait()
    c0 = coef_v[0, pl.ds(0, L)]
    c1 = coef_v[1, pl.ds(0, L)]

    def ring_body(i, _):
        g = i * NSLOT
        for s in range(NSLOT):
            gg = g + s

            @pl.when(gg + NSLOT - 1 < NCHUNK)
            def _():
                start(gg + NSLOT - 1, (s + NSLOT - 1) % NSLOT)

            wait(gg, s)
            compute(gg, s)
        return 0

    lax.fori_loop(0, NCHUNK // NSLOT, ring_body, 0)

    pltpu.sync_copy(ot_buf, o_tro.at[pl.ds(base, ROWS_PER_W)])
    pltpu.sync_copy(oa_buf, o_act.at[pl.ds(base, ROWS_PER_W)])
    pltpu.sync_copy(oe_buf, o_ema.at[pl.ds(base, ROWS_PER_W)])
    pltpu.sync_copy(og_buf, o_gain.at[pl.ds(base, ROWS_PER_W)])


@jax.jit
def _run(activities, src_idx, dst_idx, trophic, ema_coact, active, coef):
    f32 = jnp.float32
    out = jax.ShapeDtypeStruct((N,), f32)
    k = pl.kernel(
        _sc_body,
        out_type=(out, out, out, out),
        mesh=plsc.VectorSubcoreMesh(core_axis_name="c", subcore_axis_name="s"),
        compiler_params=pltpu.CompilerParams(needs_layout_passes=False),
        scratch_types=(
            pltpu.VMEM((ROWS_PER_W,), jnp.int32),
            pltpu.VMEM((ROWS_PER_W,), jnp.int32),
            pltpu.VMEM((C, B), f32),
            pltpu.VMEM((C, B), f32),
            pltpu.VMEM((C, B), f32),
            pltpu.VMEM((C, B), f32),
            pltpu.VMEM((C, B), f32),
            pltpu.VMEM((C, B), f32),
            pltpu.VMEM((C, B), f32),
            pltpu.VMEM((C, B), f32),
            pltpu.VMEM((C,), f32),
            pltpu.VMEM((ROWS_PER_W,), f32),
            pltpu.VMEM((ROWS_PER_W,), f32),
            pltpu.VMEM((ROWS_PER_W,), f32),
            pltpu.VMEM((ROWS_PER_W,), f32),
            pltpu.VMEM((ROWS_PER_W,), f32),
            pltpu.VMEM((ROWS_PER_W,), f32),
            pltpu.VMEM((ROWS_PER_W,), f32),
            pltpu.VMEM((2, L), f32),
            pltpu.SemaphoreType.DMA,
            pltpu.SemaphoreType.DMA,
            pltpu.SemaphoreType.DMA,
            pltpu.SemaphoreType.DMA,
            pltpu.SemaphoreType.DMA,
            pltpu.SemaphoreType.DMA,
            pltpu.SemaphoreType.DMA,
            pltpu.SemaphoreType.DMA,
            pltpu.SemaphoreType.DMA,
        ),
    )
    return k(activities, src_idx, dst_idx, trophic, ema_coact, active, coef)


def kernel(activities, src_idx, dst_idx, trophic, ema_coact, active,
           bdnf, ngf):
    bdnf_eff = jnp.clip(bdnf[0] * 0.05, 0.0, 0.05)
    ngf_eff = jnp.clip(ngf[0] * 0.01, 0.0, 0.01)
    g = bdnf_eff + 0.005
    c0 = 0.1 * g - (ngf_eff + 0.003)
    c1 = g + 0.001
    coef = jnp.stack([jnp.broadcast_to(c0, (L,)),
                      jnp.broadcast_to(c1, (L,))]).astype(jnp.float32)
    return _run(activities, src_idx, dst_idx, trophic, ema_coact, active,
                coef)
